# baseline (speedup 1.0000x reference)
import functools
import math

import jax
import jax.numpy as jnp
from jax import lax
from jax.experimental import pallas as pl
from jax.experimental.pallas import tpu as pltpu
from jax.experimental.pallas import tpu_sc as plsc

EMBED_DIM = 32
SCALE = math.sqrt(EMBED_DIM)

NUM_CORES = 2
NUM_SUBCORES = 16
NUM_WORKERS = NUM_CORES * NUM_SUBCORES

RW = 128
NWIN = 4
C0 = 5
NCHUNK = 10


@functools.lru_cache(maxsize=None)
def _build_detile(num_rows: int, num_cols: int):
    n_tiles_c = (num_cols + 7) // 8
    n_win = num_rows // RW
    win_per_worker = n_win // NUM_WORKERS
    mesh = plsc.VectorSubcoreMesh(core_axis_name="c", subcore_axis_name="s")

    @functools.partial(
        pl.kernel,
        mesh=mesh,
        out_type=jax.ShapeDtypeStruct((num_rows * num_cols,), jnp.int32),
        scratch_types=[
            pltpu.VMEM((n_tiles_c * 8, RW), jnp.int32),
            pltpu.SemaphoreType.DMA,
            pltpu.SemaphoreType.DMA,
        ],
        compiler_params=pltpu.CompilerParams(
            use_tc_tiling_on_sc=True, needs_layout_passes=False
        ),
    )
    def detile(xt_hbm, out_hbm, stag, isem, osem):
        wid = lax.axis_index("s") * NUM_CORES + lax.axis_index("c")

        def win(k, carry):
            wdg = wid * win_per_worker + k
            r0 = wdg * RW
            for q in range(n_tiles_c):
                h = min(8, num_cols - q * 8)
                pltpu.async_copy(
                    xt_hbm.at[pl.ds(q * 8, h), pl.ds(r0, RW)],
                    stag.at[pl.ds(q * 8, h)],
                    isem,
                )
            for q in range(n_tiles_c):
                h = min(8, num_cols - q * 8)
                pltpu.make_async_copy(
                    xt_hbm.at[pl.ds(q * 8, h), pl.ds(r0, RW)],
                    stag.at[pl.ds(q * 8, h)],
                    isem,
                ).wait()
            for c in range(num_cols):
                pltpu.async_copy(
                    stag.at[c],
                    out_hbm.at[pl.ds((wdg * num_cols + c) * RW, RW)],
                    osem,
                )
            for c in range(num_cols):
                pltpu.make_async_copy(
                    stag.at[c],
                    out_hbm.at[pl.ds((wdg * num_cols + c) * RW, RW)],
                    osem,
                ).wait()
            return carry

        lax.fori_loop(0, win_per_worker, win, 0)

    return detile


@functools.lru_cache(maxsize=None)
def _build_lookup(num_rows: int, num_cols: int):
    assert num_rows == NUM_WORKERS * NWIN * RW
    assert num_cols == C0 * NCHUNK
    slab = num_cols * RW
    mesh = plsc.VectorSubcoreMesh(core_axis_name="c", subcore_axis_name="s")

    @functools.partial(
        pl.kernel,
        mesh=mesh,
        out_type=jax.ShapeDtypeStruct(
            (num_cols, EMBED_DIM, num_rows), jnp.float32
        ),
        scratch_types=[
            pltpu.VMEM((slab,), jnp.int32),
            pltpu.VMEM((2, C0 * RW, EMBED_DIM), jnp.float32),
            pltpu.VMEM((C0, EMBED_DIM, RW), jnp.float32),
            pltpu.SemaphoreType.DMA,
            pltpu.SemaphoreType.DMA((2,)),
            pltpu.SemaphoreType.DMA,
        ],
        compiler_params=pltpu.CompilerParams(
            use_tc_tiling_on_sc=False, needs_layout_passes=False
        ),
    )
    def emb(idx_hbm, table_hbm, out_hbm, idx_v, rows_v, obuf_v, isem, gsem, osem):
        wid = lax.axis_index("s") * NUM_CORES + lax.axis_index("c")
        iota16 = lax.iota(jnp.int32, 16)

        def fire_gathers(cc, b):
            for c in range(C0):
                pltpu.async_copy(
                    table_hbm.at[idx_v.at[pl.ds((cc * C0 + c) * RW, RW)]],
                    rows_v.at[b, pl.ds(c * RW, RW)],
                    gsem.at[b],
                )

        def wait_gathers(b):
            pltpu.make_async_copy(
                table_hbm.at[pl.ds(0, C0 * RW)], rows_v.at[b], gsem.at[b]
            ).wait()

        def fire_out(cc, wdg):
            for c in range(C0):
                pltpu.async_copy(
                    obuf_v.at[c],
                    out_hbm.at[
                        cc * C0 + c,
                        pl.ds(0, EMBED_DIM),
                        pl.ds(wdg * RW, RW),
                    ],
                    osem,
                )

        def wait_out():
            pltpu.make_async_copy(
                out_hbm.at[pl.ds(0, C0), pl.ds(0, EMBED_DIM), pl.ds(0, RW)],
                obuf_v,
                osem,
            ).wait()

        cvs = [jnp.full((16,), c, jnp.int32) for c in range(C0)]
        bvs = [jnp.full((16,), b, jnp.int32) for b in range(2)]

        def transpose_scale(b):
            for c in range(C0):
                @pl.loop(0, RW // 16)
                def _(rg):
                    ovec = iota16 + rg * 16
                    rvec = ovec + c * RW
                    for h in range(EMBED_DIM // 16):
                        @plsc.parallel_loop(0, 16, unroll=8)
                        def _(d):
                            evec = lax.bitwise_and(iota16 + d, 15) + h * 16
                            v = plsc.load_gather(
                                rows_v, [bvs[b], rvec, evec]
                            ) * SCALE
                            plsc.store_scatter(
                                obuf_v, [cvs[c], evec, ovec], v
                            )

        def window(k, carry):
            wdg = wid * NWIN + k
            pltpu.async_copy(idx_hbm.at[pl.ds(wdg * slab, slab)], idx_v, isem)
            pltpu.make_async_copy(
                idx_hbm.at[pl.ds(0, slab)], idx_v, isem
            ).wait()
            fire_gathers(0, 0)

            def cpair(s, carry2):
                for b in range(2):
                    cc = s * 2 + b
                    if b == 0:
                        fire_gathers(cc + 1, 1)
                    else:
                        @pl.when(s < NCHUNK // 2 - 1)
                        def _():
                            fire_gathers(cc + 1, 0)
                    wait_gathers(b)
                    if b == 0:
                        @pl.when((k > 0) | (s > 0))
                        def _():
                            wait_out()
                    else:
                        wait_out()
                    transpose_scale(b)
                    fire_out(cc, wdg)
                return carry2

            lax.fori_loop(0, NCHUNK // 2, cpair, 0)
            return carry

        lax.fori_loop(0, NWIN, window, 0)
        wait_out()

    return emb


def kernel(x, table):
    num_rows, num_cols = x.shape
    xt = jnp.swapaxes(x, 0, 1).astype(jnp.int32)
    idx_lin = _build_detile(num_rows, num_cols)(xt)
    pout = _build_lookup(num_rows, num_cols)(idx_lin, table)
    return jnp.transpose(pout, (2, 0, 1))
